# baseline (speedup 1.0000x reference)
import jax
import jax.numpy as jnp
from jax import lax
from jax.experimental import pallas as pl
from jax.experimental.pallas import tpu as pltpu


def _p1_body(x_ref, ssq_ref, win_ref, *, kz):
    k = pl.program_id(1)
    t = x_ref[0]
    s = jnp.sum(t * t, axis=0)
    s = jnp.sum(s, axis=0)
    iz = lax.broadcasted_iota(jnp.int32, s.shape, 0)
    iy = lax.broadcasted_iota(jnp.int32, s.shape, 1) % 2
    ix = lax.broadcasted_iota(jnp.int32, s.shape, 2) % 2
    lane = lax.broadcasted_iota(jnp.int32, (1, 8), 1)
    vec = jnp.zeros((1, 8), jnp.float32)
    for p in range(8):
        pz, px, py = (p >> 2) & 1, (p >> 1) & 1, p & 1
        m = (iz == pz) & (iy == py) & (ix == px)
        v = jnp.sum(jnp.where(m, s, 0.0))
        vec = vec + jnp.where(lane == p, v, 0.0)

    @pl.when(k == 0)
    def _():
        ssq_ref[0] = vec

    @pl.when(k != 0)
    def _():
        ssq_ref[0] = ssq_ref[0] + vec

    @pl.when(k == kz - 1)
    def _():
        a = ssq_ref[0]
        mx = jnp.max(a)
        idx = jnp.min(jnp.where(a >= mx, lane, 8))
        win_ref[0] = jnp.zeros((1, 8), jnp.int32) + idx


def _p2_body(w_ref, x_ref, o_ref):
    b = pl.program_id(0)
    p = w_ref[b]
    pz = p // 4
    px = (p // 2) % 2
    py = p % 2
    t5 = x_ref[0]
    t = jnp.where(pz == 0, t5[:, :, 0], t5[:, :, 1])
    c_, zc2, yy, xx = t.shape
    rx = lax.broadcasted_iota(jnp.int32, (xx, xx // 2), 0)
    cx = lax.broadcasted_iota(jnp.int32, (xx, xx // 2), 1)
    sx = (rx == 2 * cx + px).astype(jnp.float32)
    ry = lax.broadcasted_iota(jnp.int32, (yy, yy // 2), 0)
    cy = lax.broadcasted_iota(jnp.int32, (yy, yy // 2), 1)
    sy = (ry == 2 * cy + py).astype(jnp.float32)
    a = lax.dot_general(t, sx, (((3,), (0,)), ((), ())),
                        precision=lax.Precision.HIGHEST,
                        preferred_element_type=jnp.float32)
    a = jnp.swapaxes(a, -1, -2)
    a = lax.dot_general(a, sy, (((3,), (0,)), ((), ())),
                        precision=lax.Precision.HIGHEST,
                        preferred_element_type=jnp.float32)
    o_ref[0] = jnp.swapaxes(a, -1, -2)


def kernel(input_to_pool):
    xin = input_to_pool
    B, C, Z, Y, X = xin.shape
    Zh, Yh = Z // 2, Y // 2
    x6 = xin.reshape(B, C, Zh, 2, Y, X)

    Zc = 4 if Zh % 4 == 0 else 1
    KZ = Zh // Zc

    ssq, win = pl.pallas_call(
        lambda xr, sr, wr: _p1_body(xr, sr, wr, kz=KZ),
        grid=(B, KZ),
        in_specs=[pl.BlockSpec((1, C, Zc, 2, Y, X),
                               lambda b, k: (b, 0, k, 0, 0, 0))],
        out_specs=[
            pl.BlockSpec((1, 1, 8), lambda b, k: (b, 0, 0)),
            pl.BlockSpec((1, 1, 8), lambda b, k: (b, 0, 0)),
        ],
        out_shape=[
            jax.ShapeDtypeStruct((B, 1, 8), jnp.float32),
            jax.ShapeDtypeStruct((B, 1, 8), jnp.int32),
        ],
        compiler_params=pltpu.CompilerParams(
            dimension_semantics=("parallel", "arbitrary"),
        ),
    )(x6)

    w = win[:, 0, 0]

    Zc2 = 2 if Zh % 2 == 0 else 1
    KZ2 = Zh // Zc2

    grid_spec = pltpu.PrefetchScalarGridSpec(
        num_scalar_prefetch=1,
        grid=(B, KZ2),
        in_specs=[
            pl.BlockSpec((1, C, Zc2, 2, Y, X),
                         lambda b, k, wr: (b, 0, k, 0, 0, 0)),
        ],
        out_specs=pl.BlockSpec((1, C, Zc2, Yh, X // 2),
                               lambda b, k, wr: (b, 0, k, 0, 0)),
    )
    out = pl.pallas_call(
        _p2_body,
        grid_spec=grid_spec,
        out_shape=jax.ShapeDtypeStruct((B, C, Zh, Yh, X // 2), jnp.float32),
        compiler_params=pltpu.CompilerParams(
            dimension_semantics=("parallel", "parallel"),
        ),
    )(w, x6)
    return out


# speedup vs baseline: 1.5188x; 1.5188x over previous
import jax
import jax.numpy as jnp
from jax import lax
from jax.experimental import pallas as pl
from jax.experimental.pallas import tpu as pltpu


def _p1_body(x_ref, ssq_ref, win_ref, *, kz):
    k = pl.program_id(1)
    t = x_ref[0]
    s = jnp.sum(t * t, axis=0)
    s = jnp.sum(s, axis=0)
    iz = lax.broadcasted_iota(jnp.int32, s.shape, 0)
    iy = lax.broadcasted_iota(jnp.int32, s.shape, 1) % 2
    ix = lax.broadcasted_iota(jnp.int32, s.shape, 2) % 2
    lane = lax.broadcasted_iota(jnp.int32, (1, 8), 1)
    vec = jnp.zeros((1, 8), jnp.float32)
    for p in range(8):
        pz, px, py = (p >> 2) & 1, (p >> 1) & 1, p & 1
        m = (iz == pz) & (iy == py) & (ix == px)
        v = jnp.sum(jnp.where(m, s, 0.0))
        vec = vec + jnp.where(lane == p, v, 0.0)

    @pl.when(k == 0)
    def _():
        ssq_ref[0] = vec

    @pl.when(k != 0)
    def _():
        ssq_ref[0] = ssq_ref[0] + vec

    @pl.when(k == kz - 1)
    def _():
        a = ssq_ref[0]
        mx = jnp.max(a)
        idx = jnp.min(jnp.where(a >= mx, lane, 8))
        win_ref[0] = jnp.zeros((1, 8), jnp.int32) + idx


def _p2_body(w_ref, x_ref, o_ref):
    b = pl.program_id(0)
    p = w_ref[b]
    pz = p // 4
    px = (p // 2) % 2
    py = p % 2
    t = x_ref[0, :, :, pz]
    c_, zc2, yy, xx = t.shape
    ixs = 2 * lax.broadcasted_iota(jnp.int32, (c_, zc2, yy, xx // 2), 3) + px
    a = jnp.take_along_axis(t, ixs, axis=3)
    a = jnp.swapaxes(a, -1, -2)
    iy = 2 * lax.broadcasted_iota(jnp.int32, (c_, zc2, xx // 2, yy // 2), 3) + py
    a = jnp.take_along_axis(a, iy, axis=3)
    o_ref[0] = jnp.swapaxes(a, -1, -2)


def kernel(input_to_pool):
    xin = input_to_pool
    B, C, Z, Y, X = xin.shape
    Zh, Yh = Z // 2, Y // 2
    x6 = xin.reshape(B, C, Zh, 2, Y, X)

    Zc = 4 if Zh % 4 == 0 else 1
    KZ = Zh // Zc

    ssq, win = pl.pallas_call(
        lambda xr, sr, wr: _p1_body(xr, sr, wr, kz=KZ),
        grid=(B, KZ),
        in_specs=[pl.BlockSpec((1, C, Zc, 2, Y, X),
                               lambda b, k: (b, 0, k, 0, 0, 0))],
        out_specs=[
            pl.BlockSpec((1, 1, 8), lambda b, k: (b, 0, 0)),
            pl.BlockSpec((1, 1, 8), lambda b, k: (b, 0, 0)),
        ],
        out_shape=[
            jax.ShapeDtypeStruct((B, 1, 8), jnp.float32),
            jax.ShapeDtypeStruct((B, 1, 8), jnp.int32),
        ],
        compiler_params=pltpu.CompilerParams(
            dimension_semantics=("parallel", "arbitrary"),
        ),
    )(x6)

    w = win[:, 0, 0]

    Zc2 = 2 if Zh % 2 == 0 else 1
    KZ2 = Zh // Zc2

    grid_spec = pltpu.PrefetchScalarGridSpec(
        num_scalar_prefetch=1,
        grid=(B, KZ2),
        in_specs=[
            pl.BlockSpec((1, C, Zc2, 2, Y, X),
                         lambda b, k, wr: (b, 0, k, 0, 0, 0)),
        ],
        out_specs=pl.BlockSpec((1, C, Zc2, Yh, X // 2),
                               lambda b, k, wr: (b, 0, k, 0, 0)),
    )
    out = pl.pallas_call(
        _p2_body,
        grid_spec=grid_spec,
        out_shape=jax.ShapeDtypeStruct((B, C, Zh, Yh, X // 2), jnp.float32),
        compiler_params=pltpu.CompilerParams(
            dimension_semantics=("parallel", "parallel"),
        ),
    )(w, x6)
    return out


# speedup vs baseline: 1.5722x; 1.0352x over previous
import jax
import jax.numpy as jnp
from jax import lax
from jax.experimental import pallas as pl
from jax.experimental.pallas import tpu as pltpu


def _p1_body(x_ref, ssq_ref, win_ref, *, kz):
    k = pl.program_id(1)
    t = x_ref[0]
    s = jnp.sum(t * t, axis=0)
    s = jnp.sum(s, axis=0)
    iz = lax.broadcasted_iota(jnp.int32, s.shape, 0)
    iy = lax.broadcasted_iota(jnp.int32, s.shape, 1) % 2
    ix = lax.broadcasted_iota(jnp.int32, s.shape, 2) % 2
    lane = lax.broadcasted_iota(jnp.int32, (1, 8), 1)
    vec = jnp.zeros((1, 8), jnp.float32)
    for p in range(8):
        pz, px, py = (p >> 2) & 1, (p >> 1) & 1, p & 1
        m = (iz == pz) & (iy == py) & (ix == px)
        v = jnp.sum(jnp.where(m, s, 0.0))
        vec = vec + jnp.where(lane == p, v, 0.0)

    @pl.when(k == 0)
    def _():
        ssq_ref[0] = vec

    @pl.when(k != 0)
    def _():
        ssq_ref[0] = ssq_ref[0] + vec

    @pl.when(k == kz - 1)
    def _():
        a = ssq_ref[0]
        mx = jnp.max(a)
        idx = jnp.min(jnp.where(a >= mx, lane, 8))
        win_ref[0] = jnp.zeros((1, 8), jnp.int32) + idx


def _p2_body(w_ref, x_ref, o_ref):
    b = pl.program_id(0)
    p = w_ref[b]
    pz = p // 4
    px = (p // 2) % 2
    py = p % 2
    t = x_ref[0, :, :, pz]
    c_, zc2, yy, xx = t.shape
    ixs = 2 * lax.broadcasted_iota(jnp.int32, (c_, zc2, yy, xx // 2), 3) + px
    a = jnp.take_along_axis(t, ixs, axis=3)
    a = jnp.swapaxes(a, -1, -2)
    iy = 2 * lax.broadcasted_iota(jnp.int32, (c_, zc2, xx // 2, yy // 2), 3) + py
    a = jnp.take_along_axis(a, iy, axis=3)
    o_ref[0] = jnp.swapaxes(a, -1, -2)


def kernel(input_to_pool):
    xin = input_to_pool
    B, C, Z, Y, X = xin.shape
    Zh, Yh = Z // 2, Y // 2
    x6 = xin.reshape(B, C, Zh, 2, Y, X)

    Zc = 4 if Zh % 4 == 0 else 1
    KZ = Zh // Zc

    ssq, win = pl.pallas_call(
        lambda xr, sr, wr: _p1_body(xr, sr, wr, kz=KZ),
        grid=(B, KZ),
        in_specs=[pl.BlockSpec((1, C, Zc, 2, Y, X),
                               lambda b, k: (b, 0, k, 0, 0, 0))],
        out_specs=[
            pl.BlockSpec((1, 1, 8), lambda b, k: (b, 0, 0)),
            pl.BlockSpec((1, 1, 8), lambda b, k: (b, 0, 0)),
        ],
        out_shape=[
            jax.ShapeDtypeStruct((B, 1, 8), jnp.float32),
            jax.ShapeDtypeStruct((B, 1, 8), jnp.int32),
        ],
        compiler_params=pltpu.CompilerParams(
            dimension_semantics=("parallel", "arbitrary"),
        ),
    )(x6)

    w = win[:, 0, 0]

    Zc2 = 4 if Zh % 4 == 0 else 1
    KZ2 = Zh // Zc2

    grid_spec = pltpu.PrefetchScalarGridSpec(
        num_scalar_prefetch=1,
        grid=(B, KZ2),
        in_specs=[
            pl.BlockSpec((1, C, Zc2, 2, Y, X),
                         lambda b, k, wr: (b, 0, k, 0, 0, 0)),
        ],
        out_specs=pl.BlockSpec((1, C, Zc2, Yh, X // 2),
                               lambda b, k, wr: (b, 0, k, 0, 0)),
    )
    out = pl.pallas_call(
        _p2_body,
        grid_spec=grid_spec,
        out_shape=jax.ShapeDtypeStruct((B, C, Zh, Yh, X // 2), jnp.float32),
        compiler_params=pltpu.CompilerParams(
            dimension_semantics=("parallel", "parallel"),
        ),
    )(w, x6)
    return out
